# Initial kernel scaffold; baseline (speedup 1.0000x reference)
import jax
import jax.numpy as jnp
from jax.experimental import pallas as pl


def kernel(x, pos, vel, edge_index, We1, be1, We2, be2, We3, be3, Wv1, bv1, Wv2, bv2, Wh1, bh1, Wh2, bh2):
    raise NotImplementedError("write your pallas kernel here")


# speedup vs baseline: 4.1705x; 4.1705x over previous
import functools

import jax
import jax.numpy as jnp
from jax import lax
from jax.experimental import pallas as pl
from jax.experimental.pallas import tpu as pltpu
from jax.experimental.pallas import tpu_sc as plsc

_NC, _NS = 2, 16
_NW = _NC * _NS
_QW = 16
_MW = 80
_CHUNK = 80


def _proj_body(x_ref, wi_ref, wj_ref, bi_ref, ti_ref, tj_ref):
    x = x_ref[...]
    ti_ref[...] = (
        jnp.dot(x, wi_ref[...], preferred_element_type=jnp.float32) + bi_ref[...]
    )
    tj_ref[...] = jnp.dot(x, wj_ref[...], preferred_element_type=jnp.float32)


def _make_gather(n, e, d2):
    epw = e // _NW
    nchunk = epw // _CHUNK
    mesh = plsc.VectorSubcoreMesh(core_axis_name="c", subcore_axis_name="s")

    @functools.partial(
        pl.kernel,
        mesh=mesh,
        compiler_params=pltpu.CompilerParams(use_tc_tiling_on_sc=False),
        out_type=(
            jax.ShapeDtypeStruct((e, d2), jnp.float32),
            jax.ShapeDtypeStruct((e, d2), jnp.float32),
            jax.ShapeDtypeStruct((e, _QW), jnp.float32),
            jax.ShapeDtypeStruct((e, _QW), jnp.float32),
        ),
        scratch_types=[
            pltpu.VMEM((_CHUNK,), jnp.int32),
            pltpu.VMEM((_CHUNK,), jnp.int32),
            pltpu.VMEM((_CHUNK, d2), jnp.float32),
            pltpu.VMEM((_CHUNK, d2), jnp.float32),
            pltpu.VMEM((_CHUNK, _QW), jnp.float32),
            pltpu.VMEM((_CHUNK, _QW), jnp.float32),
            pltpu.SemaphoreType.DMA,
        ],
    )
    def gather_k(src_hbm, dst_hbm, ti_hbm, tj_hbm, q_hbm,
                 gi_hbm, gj_hbm, qi_hbm, qj_hbm,
                 di_v, sj_v, gi_v, gj_v, qi_v, qj_v, sem):
        wid = lax.axis_index("c") * _NS + lax.axis_index("s")
        base0 = wid * epw

        def body(k, carry):
            base = base0 + k * _CHUNK
            pltpu.sync_copy(dst_hbm.at[pl.ds(base, _CHUNK)], di_v)
            pltpu.sync_copy(src_hbm.at[pl.ds(base, _CHUNK)], sj_v)
            c1 = pltpu.async_copy(ti_hbm.at[di_v], gi_v, sem)
            c2 = pltpu.async_copy(tj_hbm.at[sj_v], gj_v, sem)
            c3 = pltpu.async_copy(q_hbm.at[di_v], qi_v, sem)
            c4 = pltpu.async_copy(q_hbm.at[sj_v], qj_v, sem)
            c1.wait()
            c2.wait()
            c3.wait()
            c4.wait()
            pltpu.sync_copy(gi_v, gi_hbm.at[pl.ds(base, _CHUNK)])
            pltpu.sync_copy(gj_v, gj_hbm.at[pl.ds(base, _CHUNK)])
            pltpu.sync_copy(qi_v, qi_hbm.at[pl.ds(base, _CHUNK)])
            pltpu.sync_copy(qj_v, qj_hbm.at[pl.ds(base, _CHUNK)])
            return carry

        lax.fori_loop(0, nchunk, body, 0)

    return gather_k


def _edge_body(gi_ref, gj_ref, qi_ref, qj_ref,
               we2_ref, be2_ref, we3_ref, be3_ref,
               weg_ref, wer_ref, wvg_ref, wvr_ref,
               wv2_ref, bv2_ref, out_ref):
    gi = gi_ref[...]
    gj = gj_ref[...]
    rel = qj_ref[...] - qi_ref[...]
    rp0 = rel[:, 0:1]
    rp1 = rel[:, 1:2]
    rv0 = rel[:, 2:3]
    rv1 = rel[:, 3:4]
    dist = rp0 * rp0 + rp1 * rp1
    dot = rv0 * rp0 + rv1 * rp1
    h = 64
    pre = gi[:, :h] + gj[:, :h] + dist * weg_ref[...] + dot * wer_ref[...]
    h1 = jax.nn.silu(pre)
    h2 = jax.nn.silu(
        jnp.dot(h1, we2_ref[...], preferred_element_type=jnp.float32) + be2_ref[...]
    )
    mh = jnp.dot(h2, we3_ref[...], preferred_element_type=jnp.float32) + be3_ref[...]
    pv = gi[:, h:] + gj[:, h:] + dist * wvg_ref[...] + dot * wvr_ref[...]
    v = jax.nn.silu(pv)
    vw = jnp.sum(v * wv2_ref[...], axis=1, keepdims=True) + bv2_ref[...]
    lane = lax.broadcasted_iota(jnp.int32, rel.shape, 1)
    mv = jnp.where(lane < 2, vw * rel, 0.0)
    out_ref[...] = jnp.concatenate([mh, mv], axis=1)


def _make_scatter(n, e):
    epw = e // _NW
    nchunk = epw // _CHUNK
    npt = n // _NS
    mesh = plsc.VectorSubcoreMesh(core_axis_name="c", subcore_axis_name="s")

    @functools.partial(
        pl.kernel,
        mesh=mesh,
        compiler_params=pltpu.CompilerParams(use_tc_tiling_on_sc=False),
        out_type=jax.ShapeDtypeStruct((_NC, n, _MW), jnp.float32),
        scratch_types=[
            pltpu.VMEM((_CHUNK,), jnp.int32),
            pltpu.VMEM((_CHUNK, _MW), jnp.float32),
            pltpu.VMEM_SHARED((n, _MW), jnp.float32),
        ],
    )
    def scatter_k(dst_hbm, me_hbm, z_hbm, acc_hbm, di_v, me_v, acc_sh):
        c = lax.axis_index("c")
        s = lax.axis_index("s")
        pltpu.sync_copy(z_hbm.at[pl.ds(s * npt, npt)],
                        acc_sh.at[pl.ds(s * npt, npt)])
        plsc.subcore_barrier()
        base0 = (c * _NS + s) * epw

        def body(k, carry):
            base = base0 + k * _CHUNK
            pltpu.sync_copy(dst_hbm.at[pl.ds(base, _CHUNK)], di_v)
            pltpu.sync_copy(me_hbm.at[pl.ds(base, _CHUNK)], me_v)
            pltpu.sync_copy(me_v, acc_sh.at[di_v], add=True)
            return carry

        lax.fori_loop(0, nchunk, body, 0)
        plsc.subcore_barrier()
        pltpu.sync_copy(acc_sh.at[pl.ds(s * npt, npt)],
                        acc_hbm.at[c, pl.ds(s * npt, npt)])

    return scatter_k


def _node_body(x_ref, a0_ref, a1_ref, whx_ref, whm_ref, whn_ref, bh1_ref,
               wh2_ref, bh2_ref, out_ref):
    a = a0_ref[...] + a1_ref[...]
    h = 64
    mh = a[:, :h]
    mv0 = a[:, h:h + 1]
    mv1 = a[:, h + 1:h + 2]
    norm = jnp.sqrt(mv0 * mv0 + mv1 * mv1 + 1e-12)
    pre = (
        jnp.dot(x_ref[...], whx_ref[...], preferred_element_type=jnp.float32)
        + jnp.dot(mh, whm_ref[...], preferred_element_type=jnp.float32)
        + norm * whn_ref[...]
        + bh1_ref[...]
    )
    out_ref[...] = (
        jnp.dot(jax.nn.silu(pre), wh2_ref[...], preferred_element_type=jnp.float32)
        + bh2_ref[...]
    )


def kernel(x, pos, vel, edge_index, We1, be1, We2, be2, We3, be3,
           Wv1, bv1, Wv2, bv2, Wh1, bh1, Wh2, bh2):
    n, d = x.shape
    e = edge_index.shape[1]
    h = We2.shape[0]
    o = Wh2.shape[1]
    d2 = 2 * h

    src = edge_index[0]
    dst = edge_index[1]

    wi = jnp.concatenate([We1[:d], Wv1[:d]], axis=1)
    wj = jnp.concatenate([We1[d:2 * d], Wv1[d:2 * d]], axis=1)
    bi = jnp.concatenate([be1, bv1])[None, :]
    weg = We1[2 * d][None, :]
    wer = We1[2 * d + 1][None, :]
    wvg = Wv1[2 * d][None, :]
    wvr = Wv1[2 * d + 1][None, :]
    q = jnp.concatenate(
        [pos, vel, jnp.zeros((n, _QW - 4), jnp.float32)], axis=1)
    zeros = jnp.zeros((n, _MW), jnp.float32)
    whx = Wh1[:d]
    whm = Wh1[d:d + h]
    whn = Wh1[d + h][None, :]

    ti, tj = pl.pallas_call(
        _proj_body,
        out_shape=(
            jax.ShapeDtypeStruct((n, d2), jnp.float32),
            jax.ShapeDtypeStruct((n, d2), jnp.float32),
        ),
    )(x, wi, wj, bi)

    gi, gj, qi, qj = _make_gather(n, e, d2)(src, dst, ti, tj, q)

    eb = 2000
    grid = e // eb
    row_spec = lambda w: pl.BlockSpec((eb, w), lambda i: (i, 0))
    full = lambda a: pl.BlockSpec(a.shape, lambda i: (0,) * a.ndim)
    me = pl.pallas_call(
        _edge_body,
        grid=(grid,),
        in_specs=[
            row_spec(d2), row_spec(d2), row_spec(_QW), row_spec(_QW),
            full(We2), pl.BlockSpec((1, h), lambda i: (0, 0)),
            full(We3), pl.BlockSpec((1, h), lambda i: (0, 0)),
            full(weg), full(wer), full(wvg), full(wvr),
            pl.BlockSpec((1, h), lambda i: (0, 0)),
            pl.BlockSpec((1, 1), lambda i: (0, 0)),
        ],
        out_specs=row_spec(_MW),
        out_shape=jax.ShapeDtypeStruct((e, _MW), jnp.float32),
    )(gi, gj, qi, qj, We2, be2[None, :], We3, be3[None, :],
      weg, wer, wvg, wvr, Wv2.T, bv2[None, :])

    acc = _make_scatter(n, e)(dst, me, zeros)

    out = pl.pallas_call(
        _node_body,
        out_shape=jax.ShapeDtypeStruct((n, o), jnp.float32),
    )(x, acc[0], acc[1], whx, whm, whn, bh1[None, :], Wh2, bh2[None, :])
    return out


# speedup vs baseline: 6.2064x; 1.4882x over previous
import functools

import jax
import jax.numpy as jnp
from jax import lax
from jax.experimental import pallas as pl
from jax.experimental.pallas import tpu as pltpu
from jax.experimental.pallas import tpu_sc as plsc

_NC, _NS = 2, 16
_NW = _NC * _NS
_QW = 16
_MW = 80
_CHUNK = 80


def _proj_body(x_ref, wi_ref, wj_ref, bi_ref, ti_ref, tj_ref):
    x = x_ref[...]
    ti_ref[...] = (
        jnp.dot(x, wi_ref[...], preferred_element_type=jnp.float32) + bi_ref[...]
    )
    tj_ref[...] = jnp.dot(x, wj_ref[...], preferred_element_type=jnp.float32)


def _make_gather(n, e, d2):
    epw = e // _NW
    nchunk = epw // _CHUNK
    mesh = plsc.VectorSubcoreMesh(core_axis_name="c", subcore_axis_name="s")

    @functools.partial(
        pl.kernel,
        mesh=mesh,
        compiler_params=pltpu.CompilerParams(
            use_tc_tiling_on_sc=False, has_side_effects=True),
        out_type=(
            jax.ShapeDtypeStruct((e, d2), jnp.float32),
            jax.ShapeDtypeStruct((e, d2), jnp.float32),
            jax.ShapeDtypeStruct((e, _QW), jnp.float32),
            jax.ShapeDtypeStruct((e, _QW), jnp.float32),
            jax.ShapeDtypeStruct((16,), jnp.float32),
        ),
        scratch_types=[
            [pltpu.VMEM((_CHUNK,), jnp.int32) for _ in range(2)],
            [pltpu.VMEM((_CHUNK,), jnp.int32) for _ in range(2)],
            [pltpu.VMEM((_CHUNK, d2), jnp.float32) for _ in range(2)],
            [pltpu.VMEM((_CHUNK, d2), jnp.float32) for _ in range(2)],
            [pltpu.VMEM((_CHUNK, _QW), jnp.float32) for _ in range(2)],
            [pltpu.VMEM((_CHUNK, _QW), jnp.float32) for _ in range(2)],
            [pltpu.SemaphoreType.DMA for _ in range(2)],
            pltpu.VMEM((16,), jnp.float32),
        ],
    )
    def gather_k(src_hbm, dst_hbm, ti_hbm, tj_hbm, q_hbm, tok_hbm,
                 gi_hbm, gj_hbm, qi_hbm, qj_hbm, tok_out,
                 di_v, sj_v, gi_v, gj_v, qi_v, qj_v, sem, tok_v):
        wid = lax.axis_index("c") * _NS + lax.axis_index("s")
        base0 = wid * epw

        @pl.when(wid == 0)
        def _():
            pltpu.sync_copy(tok_hbm, tok_v)
            pltpu.sync_copy(tok_v, tok_out)

        def start(k, b):
            base = base0 + k * _CHUNK
            pltpu.sync_copy(dst_hbm.at[pl.ds(base, _CHUNK)], di_v[b])
            pltpu.sync_copy(src_hbm.at[pl.ds(base, _CHUNK)], sj_v[b])
            pltpu.async_copy(ti_hbm.at[di_v[b]], gi_v[b], sem[b])
            pltpu.async_copy(tj_hbm.at[sj_v[b]], gj_v[b], sem[b])
            pltpu.async_copy(q_hbm.at[di_v[b]], qi_v[b], sem[b])
            pltpu.async_copy(q_hbm.at[sj_v[b]], qj_v[b], sem[b])

        def finish(k, b):
            base = base0 + k * _CHUNK
            pltpu.make_async_copy(ti_hbm.at[di_v[b]], gi_v[b], sem[b]).wait()
            pltpu.make_async_copy(tj_hbm.at[sj_v[b]], gj_v[b], sem[b]).wait()
            pltpu.make_async_copy(q_hbm.at[di_v[b]], qi_v[b], sem[b]).wait()
            pltpu.make_async_copy(q_hbm.at[sj_v[b]], qj_v[b], sem[b]).wait()
            pltpu.sync_copy(gi_v[b], gi_hbm.at[pl.ds(base, _CHUNK)])
            pltpu.sync_copy(gj_v[b], gj_hbm.at[pl.ds(base, _CHUNK)])
            pltpu.sync_copy(qi_v[b], qi_hbm.at[pl.ds(base, _CHUNK)])
            pltpu.sync_copy(qj_v[b], qj_hbm.at[pl.ds(base, _CHUNK)])

        start(0, 0)

        def body(t, carry):
            k0 = 2 * t
            k1 = k0 + 1
            k2 = k0 + 2

            @pl.when(k1 < nchunk)
            def _():
                start(k1, 1)

            finish(k0, 0)

            @pl.when(k2 < nchunk)
            def _():
                start(k2, 0)

            @pl.when(k1 < nchunk)
            def _():
                finish(k1, 1)

            return carry

        lax.fori_loop(0, (nchunk + 1) // 2, body, 0)

    return gather_k


def _edge_body(gi_ref, gj_ref, qi_ref, qj_ref,
               pg_ref, mg_ref, we2p_ref, be2_ref, we3_ref, be3_ref,
               wv2p_ref, bv2_ref, mask_ref, out_ref):
    u = qj_ref[...] - qi_ref[...]
    w = u * jnp.dot(u, pg_ref[...], preferred_element_type=jnp.float32)
    gsum = (gi_ref[...] + gj_ref[...]
            + jnp.dot(w, mg_ref[...], preferred_element_type=jnp.float32))
    s = jax.nn.silu(gsum)
    h2 = jax.nn.silu(
        jnp.dot(s, we2p_ref[...], preferred_element_type=jnp.float32)
        + be2_ref[...]
    )
    mh = jnp.dot(h2, we3_ref[...], preferred_element_type=jnp.float32) + be3_ref[...]
    vw = (jnp.dot(s, wv2p_ref[...], preferred_element_type=jnp.float32)
          + bv2_ref[...])
    mv = vw * u * mask_ref[...]
    out_ref[...] = jnp.concatenate([mh, mv], axis=1)


def _make_scatter(n, e):
    epw = e // _NW
    nchunk = epw // _CHUNK
    npt = n // _NS
    mesh = plsc.VectorSubcoreMesh(core_axis_name="c", subcore_axis_name="s")

    @functools.partial(
        pl.kernel,
        mesh=mesh,
        compiler_params=pltpu.CompilerParams(
            use_tc_tiling_on_sc=False, has_side_effects=True),
        out_type=jax.ShapeDtypeStruct((_NC, n, _MW), jnp.float32),
        scratch_types=[
            [pltpu.VMEM((_CHUNK,), jnp.int32) for _ in range(2)],
            [pltpu.VMEM((_CHUNK, _MW), jnp.float32) for _ in range(2)],
            pltpu.VMEM_SHARED((n, _MW), jnp.float32),
            [pltpu.SemaphoreType.DMA for _ in range(2)],
            pltpu.VMEM((16,), jnp.float32),
        ],
    )
    def scatter_k(dst_hbm, me_hbm, z_hbm, tok_hbm, acc_hbm,
                  di_v, me_v, acc_sh, sem, tok_v):
        c = lax.axis_index("c")
        s = lax.axis_index("s")

        @pl.when((c == 0) & (s == 0))
        def _():
            pltpu.sync_copy(tok_hbm, tok_v)
        pltpu.sync_copy(z_hbm.at[c, pl.ds(s * npt, npt)],
                        acc_sh.at[pl.ds(s * npt, npt)])
        plsc.subcore_barrier()
        base0 = (c * _NS + s) * epw

        def start(k, b):
            base = base0 + k * _CHUNK
            pltpu.async_copy(dst_hbm.at[pl.ds(base, _CHUNK)], di_v[b], sem[b])
            pltpu.async_copy(me_hbm.at[pl.ds(base, _CHUNK)], me_v[b], sem[b])

        def finish(k, b):
            base = base0 + k * _CHUNK
            pltpu.make_async_copy(dst_hbm.at[pl.ds(base, _CHUNK)],
                                  di_v[b], sem[b]).wait()
            pltpu.make_async_copy(me_hbm.at[pl.ds(base, _CHUNK)],
                                  me_v[b], sem[b]).wait()
            pltpu.sync_copy(me_v[b], acc_sh.at[di_v[b]], add=True)

        start(0, 0)

        def body(t, carry):
            k0 = 2 * t
            k1 = k0 + 1
            k2 = k0 + 2

            @pl.when(k1 < nchunk)
            def _():
                start(k1, 1)

            finish(k0, 0)

            @pl.when(k2 < nchunk)
            def _():
                start(k2, 0)

            @pl.when(k1 < nchunk)
            def _():
                finish(k1, 1)

            return carry

        lax.fori_loop(0, (nchunk + 1) // 2, body, 0)
        plsc.subcore_barrier()
        pltpu.sync_copy(acc_sh.at[pl.ds(s * npt, npt)],
                        acc_hbm.at[c, pl.ds(s * npt, npt)])

    return scatter_k


def _node_body(x_ref, a0_ref, a1_ref, whx_ref, whm_ref, whn_ref, bh1_ref,
               wh2_ref, bh2_ref, out_ref):
    a = a0_ref[...] + a1_ref[...]
    h = 64
    mh = a[:, :h]
    mv0 = a[:, h:h + 1]
    mv1 = a[:, h + 1:h + 2]
    norm = jnp.sqrt(mv0 * mv0 + mv1 * mv1 + 1e-12)
    pre = (
        jnp.dot(x_ref[...], whx_ref[...], preferred_element_type=jnp.float32)
        + jnp.dot(mh, whm_ref[...], preferred_element_type=jnp.float32)
        + norm * whn_ref[...]
        + bh1_ref[...]
    )
    out_ref[...] = (
        jnp.dot(jax.nn.silu(pre), wh2_ref[...], preferred_element_type=jnp.float32)
        + bh2_ref[...]
    )


def kernel(x, pos, vel, edge_index, We1, be1, We2, be2, We3, be3,
           Wv1, bv1, Wv2, bv2, Wh1, bh1, Wh2, bh2):
    n, d = x.shape
    e = edge_index.shape[1]
    h = We2.shape[0]
    o = Wh2.shape[1]
    d2 = 2 * h

    src = edge_index[0]
    dst = edge_index[1]

    wi = jnp.concatenate([We1[:d], Wv1[:d]], axis=1)
    wj = jnp.concatenate([We1[d:2 * d], Wv1[d:2 * d]], axis=1)
    bi = jnp.concatenate([be1, bv1])[None, :]
    pg = (jnp.zeros((_QW, _QW), jnp.float32)
          .at[0, 0].set(1.0).at[1, 1].set(1.0)
          .at[0, 2].set(1.0).at[1, 3].set(1.0))
    row_g = jnp.concatenate([We1[2 * d], Wv1[2 * d]])
    row_r = jnp.concatenate([We1[2 * d + 1], Wv1[2 * d + 1]])
    mg = (jnp.zeros((_QW, d2), jnp.float32)
          .at[0].set(row_g).at[1].set(row_g)
          .at[2].set(row_r).at[3].set(row_r))
    we2p = jnp.concatenate([We2, jnp.zeros((h, h), jnp.float32)], axis=0)
    wv2p = jnp.concatenate(
        [jnp.zeros((h, _QW), jnp.float32), jnp.tile(Wv2, (1, _QW))], axis=0)
    lane_mask = (jnp.arange(_QW) < 2).astype(jnp.float32)[None, :]
    q = jnp.concatenate(
        [pos, vel, jnp.zeros((n, _QW - 4), jnp.float32)], axis=1)
    zeros = jnp.zeros((_NC, n, _MW), jnp.float32)
    whx = Wh1[:d]
    whm = Wh1[d:d + h]
    whn = Wh1[d + h][None, :]

    ti, tj = pl.pallas_call(
        _proj_body,
        out_shape=(
            jax.ShapeDtypeStruct((n, d2), jnp.float32),
            jax.ShapeDtypeStruct((n, d2), jnp.float32),
        ),
    )(x, wi, wj, bi)

    npipe = 5
    ec = e // npipe
    gather_k = _make_gather(n, ec, d2)
    scatter_k = _make_scatter(n, ec)

    eb = 2000
    grid = ec // eb
    row_spec = lambda w: pl.BlockSpec((eb, w), lambda i: (i, 0))
    full = lambda a: pl.BlockSpec(a.shape, lambda i: (0,) * a.ndim)
    edge_mlp = pl.pallas_call(
        _edge_body,
        grid=(grid,),
        in_specs=[
            row_spec(d2), row_spec(d2), row_spec(_QW), row_spec(_QW),
            full(pg), full(mg), full(we2p),
            pl.BlockSpec((1, h), lambda i: (0, 0)),
            full(We3), pl.BlockSpec((1, h), lambda i: (0, 0)),
            full(wv2p), pl.BlockSpec((1, 1), lambda i: (0, 0)),
            full(lane_mask),
        ],
        out_specs=row_spec(_MW),
        out_shape=jax.ShapeDtypeStruct((ec, _MW), jnp.float32),
    )

    tok = jnp.zeros((16,), jnp.float32)
    gouts = []
    for ci in range(npipe):
        lo, hi = ci * ec, (ci + 1) * ec
        gi, gj, qi, qj, tok = gather_k(src[lo:hi], dst[lo:hi], ti, tj, q, tok)
        gouts.append((gi, gj, qi, qj))
    mes = [edge_mlp(gi, gj, qi, qj, pg, mg, we2p, be2[None, :],
                    We3, be3[None, :], wv2p, bv2[None, :], lane_mask)
           for (gi, gj, qi, qj) in gouts]
    acc = zeros
    for ci in range(npipe):
        lo, hi = ci * ec, (ci + 1) * ec
        acc = scatter_k(dst[lo:hi], mes[ci], acc, tok)

    out = pl.pallas_call(
        _node_body,
        out_shape=jax.ShapeDtypeStruct((n, o), jnp.float32),
    )(x, acc[0], acc[1], whx, whm, whn, bh1[None, :], Wh2, bh2[None, :])
    return out
